# baseline (speedup 1.0000x reference)
import functools

import numpy as np
import jax
import jax.numpy as jnp
from jax import lax
from jax.experimental import pallas as pl
from jax.experimental.pallas import tpu as pltpu
from jax.experimental.pallas import tpu_sc as plsc

_D = 64
_L = 200
_B = 1024
_NC = 2
_NS = 16
_NW = _NC * _NS


def _pos_encoding(length, depth):
    positions = np.arange(length).reshape(-1, 1)
    depths = np.array([2 * (i // 2) for i in range(depth)]).reshape(1, -1)
    angle_rates = 1.0 / 10000 ** (depths / depth)
    angles = positions * angle_rates
    encoding = np.cos(angles)
    encoding[:, ::2] = np.sin(encoding[:, ::2])
    return encoding.astype(np.float32)


_ENC = jnp.asarray(_pos_encoding(_L, _D))

_EPW = _B // _NW


_mesh = plsc.VectorSubcoreMesh(core_axis_name="c", subcore_axis_name="s")


@functools.partial(
    pl.kernel,
    mesh=_mesh,
    out_type=jax.ShapeDtypeStruct((_B * _L, _D), jnp.float32),
    scratch_types=[
        pltpu.VMEM((_L, _D), jnp.float32),
        pltpu.VMEM((104,), jnp.int32),
        pltpu.VMEM((96,), jnp.int32),
        pltpu.VMEM((_L, _D), jnp.float32),
        pltpu.SemaphoreType.DMA,
    ],
    compiler_params=pltpu.CompilerParams(use_tc_tiling_on_sc=False),
)
def _emb_kernel(table_hbm, xf_hbm, enc_hbm, out_hbm, enc_v, idx_a, idx_b, rows_v, sem):
    wid = lax.axis_index("s") * _NC + lax.axis_index("c")
    pltpu.sync_copy(enc_hbm, enc_v)

    def elem(e, carry):
        r0 = (wid * _EPW + e) * _L
        pltpu.sync_copy(enc_hbm, rows_v)
        pltpu.sync_copy(xf_hbm.at[pl.ds(r0, 104)], idx_a)
        pltpu.sync_copy(xf_hbm.at[pl.ds(r0 + 104, 96)], idx_b)
        cp1 = pltpu.async_copy(table_hbm.at[idx_a], rows_v.at[pl.ds(0, 104)], sem, add=True)
        cp2 = pltpu.async_copy(table_hbm.at[idx_b], rows_v.at[pl.ds(104, 96)], sem, add=True)
        cp1.wait()
        cp2.wait()
        pltpu.sync_copy(rows_v, out_hbm.at[pl.ds(r0, _L)])
        return carry

    lax.fori_loop(0, _EPW, elem, 0)


@jax.jit
def kernel(x, table):
    xf = x.reshape(-1).astype(jnp.int32)
    out = _emb_kernel(table, xf, _ENC)
    return out.reshape(_B, _L, _D)


# speedup vs baseline: 1.1518x; 1.1518x over previous
import functools

import numpy as np
import jax
import jax.numpy as jnp
from jax import lax
from jax.experimental import pallas as pl
from jax.experimental.pallas import tpu as pltpu
from jax.experimental.pallas import tpu_sc as plsc

_D = 64
_L = 200
_B = 1024
_NC = 2
_NS = 16
_NW = _NC * _NS
_EPW = _B // _NW
_NBUF = 8
_GROUPS = _EPW // _NBUF
_CA = 104
_CB = _L - _CA


def _pos_encoding(length, depth):
    positions = np.arange(length).reshape(-1, 1)
    depths = np.array([2 * (i // 2) for i in range(depth)]).reshape(1, -1)
    angle_rates = 1.0 / 10000 ** (depths / depth)
    angles = positions * angle_rates
    encoding = np.cos(angles)
    encoding[:, ::2] = np.sin(encoding[:, ::2])
    return encoding.astype(np.float32)


_ENC = jnp.asarray(_pos_encoding(_L, _D))

_mesh = plsc.VectorSubcoreMesh(core_axis_name="c", subcore_axis_name="s")


@functools.partial(
    pl.kernel,
    mesh=_mesh,
    out_type=jax.ShapeDtypeStruct((_B * _L, _D), jnp.float32),
    scratch_types=[
        pltpu.VMEM((_EPW * _L,), jnp.int32),
        [pltpu.VMEM((_L, _D), jnp.float32) for _ in range(_NBUF)],
        pltpu.VMEM_SHARED((_L, _D), jnp.float32),
        pltpu.SemaphoreType.DMA((_NBUF,)),
        pltpu.SemaphoreType.DMA((_NBUF,)),
        pltpu.SemaphoreType.DMA((_NBUF,)),
    ],
    compiler_params=pltpu.CompilerParams(use_tc_tiling_on_sc=False),
)
def _emb_kernel(table_hbm, xf_hbm, enc_hbm, out_hbm,
                idx_all, rows, enc_sh, pre_sem, g_sem, st_sem):
    sid = lax.axis_index("s")
    wid = sid * _NC + lax.axis_index("c")
    base_row = wid * _EPW * _L

    pltpu.sync_copy(xf_hbm.at[pl.ds(base_row, _EPW * _L)], idx_all)

    @pl.when(sid == 0)
    def _stage_enc():
        pltpu.sync_copy(enc_hbm, enc_sh)

    plsc.subcore_barrier()

    def pre_issue(b):
        pltpu.async_copy(enc_sh, rows[b], pre_sem.at[b])

    def pre_wait(b):
        pltpu.make_async_copy(enc_sh, rows[b], pre_sem.at[b]).wait()

    def g_issue(le, b):
        o = le * _L
        pltpu.async_copy(table_hbm.at[idx_all.at[pl.ds(o, _CA)]],
                         rows[b].at[pl.ds(0, _CA)], g_sem.at[b], add=True)
        pltpu.async_copy(table_hbm.at[idx_all.at[pl.ds(o + _CA, _CB)]],
                         rows[b].at[pl.ds(_CA, _CB)], g_sem.at[b], add=True)

    def g_wait(b):
        pltpu.make_async_copy(table_hbm.at[idx_all.at[pl.ds(0, _CA)]],
                              rows[b].at[pl.ds(0, _CA)], g_sem.at[b]).wait()
        pltpu.make_async_copy(table_hbm.at[idx_all.at[pl.ds(_CA, _CB)]],
                              rows[b].at[pl.ds(_CA, _CB)], g_sem.at[b]).wait()

    def st_issue(le, b):
        pltpu.async_copy(rows[b], out_hbm.at[pl.ds(base_row + le * _L, _L)],
                         st_sem.at[b])

    def st_wait(b):
        pltpu.make_async_copy(rows[b], out_hbm.at[pl.ds(0, _L)],
                              st_sem.at[b]).wait()

    for b in range(_NBUF):
        pre_issue(b)
    for b in range(_NBUF // 2):
        pre_wait(b)
        g_issue(b, b)

    def group(g, carry):
        le0 = _NBUF * g
        for b in range(4):
            g_wait(b)
            st_issue(le0 + b, b)
        for b in range(4, 8):
            pre_wait(b)
            g_issue(le0 + b, b)
        for b in range(4):
            st_wait(b)
            pre_issue(b)
        for b in range(4, 8):
            g_wait(b)
            st_issue(le0 + b, b)

        @pl.when(g < _GROUPS - 1)
        def _next_gathers():
            for b in range(4):
                pre_wait(b)
                g_issue(le0 + _NBUF + b, b)

        for b in range(4, 8):
            st_wait(b)
            pre_issue(b)
        return carry

    lax.fori_loop(0, _GROUPS, group, 0)

    for b in range(_NBUF):
        pre_wait(b)


@jax.jit
def kernel(x, table):
    xf = x.reshape(-1).astype(jnp.int32)
    out = _emb_kernel(table, xf, _ENC)
    return out.reshape(_B, _L, _D)
